# baseline (speedup 1.0000x reference)
import functools

import jax
import jax.numpy as jnp
from jax.experimental import pallas as pl
from jax.experimental.pallas import tpu as pltpu

D_MODEL = 2048
N_EXP = 16
TOP_K = 2
TILE = 1024


def _epilogue(logits):
    m = jnp.max(logits, axis=0, keepdims=True)
    e = jnp.exp(logits - m)
    probs = e / jnp.sum(e, axis=0, keepdims=True)
    row = jax.lax.broadcasted_iota(jnp.int32, logits.shape, 0)
    idx1 = jnp.min(jnp.where(logits == m, row, N_EXP), axis=0, keepdims=True)
    mask1 = row == idx1
    l2 = jnp.where(mask1, -jnp.inf, logits)
    m2 = jnp.max(l2, axis=0, keepdims=True)
    idx2 = jnp.min(jnp.where(l2 == m2, row, N_EXP), axis=0, keepdims=True)
    mask = mask1 | (row == idx2)
    return mask, probs


def _router_kernel(ha_ref, hb_ref, w_ref,
                   mask_a, probs_a, logits_a, mask_b, probs_b, logits_b):
    w = w_ref[...]
    la = jax.lax.dot_general(
        w, ha_ref[...], (((1,), (1,)), ((), ())),
        preferred_element_type=jnp.float32)
    lb = jax.lax.dot_general(
        w, hb_ref[...], (((1,), (1,)), ((), ())),
        preferred_element_type=jnp.float32)
    ma, pa = _epilogue(la)
    mb, pb = _epilogue(lb)
    mask_a[...] = ma
    probs_a[...] = pa
    logits_a[...] = la
    mask_b[...] = mb
    probs_b[...] = pb
    logits_b[...] = lb


@functools.partial(jax.jit, static_argnames=())
def kernel(h, W):
    n_tok = h.shape[0]
    half_tiles = n_tok // TILE // 2
    grid = (half_tiles,)
    half = n_tok // 2
    out_shape = (
        jax.ShapeDtypeStruct((N_EXP, half), jnp.bool_),
        jax.ShapeDtypeStruct((N_EXP, half), jnp.float32),
        jax.ShapeDtypeStruct((N_EXP, half), jnp.float32),
    ) * 2
    out_spec = pl.BlockSpec((N_EXP, TILE), lambda i: (0, i))
    ma, pa, la, mb, pb, lb = pl.pallas_call(
        _router_kernel,
        grid=grid,
        in_specs=[
            pl.BlockSpec((TILE, D_MODEL), lambda i: (i, 0)),
            pl.BlockSpec((TILE, D_MODEL), lambda i: (i + half_tiles, 0)),
            pl.BlockSpec((N_EXP, D_MODEL), lambda i: (0, 0)),
        ],
        out_specs=(out_spec,) * 6,
        out_shape=out_shape,
        compiler_params=pltpu.CompilerParams(
            dimension_semantics=("parallel",),
        ),
    )(h, h, W)
    mask = jnp.concatenate([ma.T, mb.T], axis=0)
    probs = jnp.concatenate([pa.T, pb.T], axis=0)
    logits = jnp.concatenate([la.T, lb.T], axis=0)
    return mask, probs, logits


# speedup vs baseline: 1.0344x; 1.0344x over previous
import functools

import jax
import jax.numpy as jnp
from jax.experimental import pallas as pl
from jax.experimental.pallas import tpu as pltpu

D_MODEL = 2048
N_EXP = 16
TOP_K = 2
CHUNK = 256
DEPTH = 8


def _epilogue(logits):
    m = jnp.max(logits, axis=0, keepdims=True)
    e = jnp.exp(logits - m)
    probs = e / jnp.sum(e, axis=0, keepdims=True)
    row = jax.lax.broadcasted_iota(jnp.int32, logits.shape, 0)
    idx1 = jnp.min(jnp.where(logits == m, row, N_EXP), axis=0, keepdims=True)
    mask1 = row == idx1
    l2 = jnp.where(mask1, -jnp.inf, logits)
    m2 = jnp.max(l2, axis=0, keepdims=True)
    idx2 = jnp.min(jnp.where(l2 == m2, row, N_EXP), axis=0, keepdims=True)
    mask = mask1 | (row == idx2)
    return mask, probs


def _router_kernel(n_chunks, h_hbm, w_ref, mask_ref, probs_ref, logits_ref,
                   buf, sem):
    i = pl.program_id(0)
    base = i * DEPTH

    def _copy(chunk_idx, slot):
        return pltpu.make_async_copy(
            h_hbm.at[pl.ds(chunk_idx * CHUNK, CHUNK), :],
            buf.at[slot],
            sem.at[slot],
        )

    @pl.when(i == 0)
    def _prologue():
        for d in range(DEPTH):
            _copy(d, d).start()

    w = w_ref[...]
    for d in range(DEPTH):
        _copy(base + d, d).wait()
        logits = jax.lax.dot_general(
            w, buf[d], (((1,), (1,)), ((), ())),
            preferred_element_type=jnp.float32,
        )
        mask, probs = _epilogue(logits)
        sl = pl.ds(d * CHUNK, CHUNK)
        mask_ref[:, sl] = mask
        probs_ref[:, sl] = probs
        logits_ref[:, sl] = logits

        @pl.when(base + d + DEPTH < n_chunks)
        def _next():
            _copy(base + d + DEPTH, d).start()


@functools.partial(jax.jit, static_argnames=())
def kernel(h, W):
    n_tok = h.shape[0]
    n_chunks = n_tok // CHUNK
    n_steps = n_chunks // DEPTH
    step_tok = DEPTH * CHUNK
    out_shapes = (
        jax.ShapeDtypeStruct((N_EXP, n_tok), jnp.bool_),
        jax.ShapeDtypeStruct((N_EXP, n_tok), jnp.float32),
        jax.ShapeDtypeStruct((N_EXP, n_tok), jnp.float32),
    )
    out_spec = pl.BlockSpec((N_EXP, step_tok), lambda i: (0, i))
    mask_t, probs_t, logits_t = pl.pallas_call(
        functools.partial(_router_kernel, n_chunks),
        grid=(n_steps,),
        in_specs=[
            pl.BlockSpec(memory_space=pltpu.MemorySpace.HBM),
            pl.BlockSpec((N_EXP, D_MODEL), lambda i: (0, 0)),
        ],
        out_specs=(out_spec, out_spec, out_spec),
        out_shape=out_shapes,
        scratch_shapes=[
            pltpu.VMEM((DEPTH, CHUNK, D_MODEL), jnp.float32),
            pltpu.SemaphoreType.DMA((DEPTH,)),
        ],
        compiler_params=pltpu.CompilerParams(
            dimension_semantics=("arbitrary",),
        ),
    )(h, W)
    return mask_t.T, probs_t.T, logits_t.T


# speedup vs baseline: 1.0560x; 1.0209x over previous
import functools

import jax
import jax.numpy as jnp
from jax.experimental import pallas as pl
from jax.experimental.pallas import tpu as pltpu

D_MODEL = 2048
N_EXP = 16
TOP_K = 2
TILE = 512


def _epilogue(logits):
    m = jnp.max(logits, axis=0, keepdims=True)
    e = jnp.exp(logits - m)
    probs = e / jnp.sum(e, axis=0, keepdims=True)
    row = jax.lax.broadcasted_iota(jnp.int32, logits.shape, 0)
    idx1 = jnp.min(jnp.where(logits == m, row, N_EXP), axis=0, keepdims=True)
    mask1 = row == idx1
    l2 = jnp.where(mask1, -jnp.inf, logits)
    m2 = jnp.max(l2, axis=0, keepdims=True)
    idx2 = jnp.min(jnp.where(l2 == m2, row, N_EXP), axis=0, keepdims=True)
    mask = mask1 | (row == idx2)
    return mask, probs


def _router_kernel(ha_ref, hb_ref, w_ref,
                   mask_a, probs_a, logits_a, mask_b, probs_b, logits_b):
    w = w_ref[...]
    la = jax.lax.dot_general(
        w, ha_ref[...], (((1,), (1,)), ((), ())),
        preferred_element_type=jnp.float32)
    lb = jax.lax.dot_general(
        w, hb_ref[...], (((1,), (1,)), ((), ())),
        preferred_element_type=jnp.float32)
    ma, pa = _epilogue(la)
    mb, pb = _epilogue(lb)
    mask_a[...] = ma
    probs_a[...] = pa
    logits_a[...] = la
    mask_b[...] = mb
    probs_b[...] = pb
    logits_b[...] = lb


@functools.partial(jax.jit, static_argnames=())
def kernel(h, W):
    n_tok = h.shape[0]
    half_tiles = n_tok // TILE // 2
    grid = (half_tiles,)
    half = n_tok // 2
    out_shape = (
        jax.ShapeDtypeStruct((N_EXP, half), jnp.bool_),
        jax.ShapeDtypeStruct((N_EXP, half), jnp.float32),
        jax.ShapeDtypeStruct((N_EXP, half), jnp.float32),
    ) * 2
    out_spec = pl.BlockSpec((N_EXP, TILE), lambda i: (0, i))
    ma, pa, la, mb, pb, lb = pl.pallas_call(
        _router_kernel,
        grid=grid,
        in_specs=[
            pl.BlockSpec((TILE, D_MODEL), lambda i: (i, 0)),
            pl.BlockSpec((TILE, D_MODEL), lambda i: (i + half_tiles, 0)),
            pl.BlockSpec((N_EXP, D_MODEL), lambda i: (0, 0)),
        ],
        out_specs=(out_spec,) * 6,
        out_shape=out_shape,
        compiler_params=pltpu.CompilerParams(
            dimension_semantics=("parallel",),
        ),
    )(h, h, W)
    mask = jnp.concatenate([ma.T, mb.T], axis=0)
    probs = jnp.concatenate([pa.T, pb.T], axis=0)
    logits = jnp.concatenate([la.T, lb.T], axis=0)
    return mask, probs, logits


# speedup vs baseline: 1.0737x; 1.0167x over previous
import functools

import jax
import jax.numpy as jnp
from jax.experimental import pallas as pl
from jax.experimental.pallas import tpu as pltpu

D_MODEL = 2048
N_EXP = 16
TOP_K = 2
TILE = 1024
NSLOT = 4


def _router_kernel(n_tiles, h_hbm, w_ref, mask_ref, probs_ref, logits_ref,
                   buf, sem):
    i = pl.program_id(0)

    def _copy(tile_idx, slot):
        return pltpu.make_async_copy(
            h_hbm.at[pl.ds(tile_idx * TILE, TILE), :],
            buf.at[slot],
            sem.at[slot],
        )

    @pl.when(i == 0)
    def _prologue():
        for d in range(NSLOT):
            _copy(d, d).start()

    slot = jax.lax.rem(i, NSLOT)
    _copy(i, slot).wait()

    w = w_ref[...]
    logits = jax.lax.dot_general(
        w, buf[slot], (((1,), (1,)), ((), ())),
        preferred_element_type=jnp.float32,
    )

    m = jnp.max(logits, axis=0, keepdims=True)
    e = jnp.exp(logits - m)
    probs = e / jnp.sum(e, axis=0, keepdims=True)

    row = jax.lax.broadcasted_iota(jnp.int32, logits.shape, 0)
    idx1 = jnp.min(jnp.where(logits == m, row, N_EXP), axis=0, keepdims=True)
    mask1 = row == idx1
    l2 = jnp.where(mask1, -jnp.inf, logits)
    m2 = jnp.max(l2, axis=0, keepdims=True)
    idx2 = jnp.min(jnp.where(l2 == m2, row, N_EXP), axis=0, keepdims=True)
    mask = mask1 | (row == idx2)

    mask_ref[...] = mask
    probs_ref[...] = probs
    logits_ref[...] = logits

    @pl.when(i + NSLOT < n_tiles)
    def _next():
        _copy(i + NSLOT, slot).start()


@functools.partial(jax.jit, static_argnames=())
def kernel(h, W):
    n_tok = h.shape[0]
    n_tiles = n_tok // TILE
    out_shapes = (
        jax.ShapeDtypeStruct((N_EXP, n_tok), jnp.bool_),
        jax.ShapeDtypeStruct((N_EXP, n_tok), jnp.float32),
        jax.ShapeDtypeStruct((N_EXP, n_tok), jnp.float32),
    )
    out_spec = pl.BlockSpec((N_EXP, TILE), lambda i: (0, i))
    mask_t, probs_t, logits_t = pl.pallas_call(
        functools.partial(_router_kernel, n_tiles),
        grid=(n_tiles,),
        in_specs=[
            pl.BlockSpec(memory_space=pltpu.MemorySpace.HBM),
            pl.BlockSpec((N_EXP, D_MODEL), lambda i: (0, 0)),
        ],
        out_specs=(out_spec, out_spec, out_spec),
        out_shape=out_shapes,
        scratch_shapes=[
            pltpu.VMEM((NSLOT, TILE, D_MODEL), jnp.float32),
            pltpu.SemaphoreType.DMA((NSLOT,)),
        ],
        compiler_params=pltpu.CompilerParams(
            dimension_semantics=("arbitrary",),
        ),
    )(h, W)
    return mask_t.T, probs_t.T, logits_t.T


# speedup vs baseline: 1.1740x; 1.0934x over previous
import functools

import jax
import jax.numpy as jnp
from jax.experimental import pallas as pl
from jax.experimental.pallas import tpu as pltpu

D_MODEL = 2048
N_EXP = 16
TOP_K = 2
TILE = 1024


def _router_kernel(h_ref, w_ref, mask_ref, probs_ref, logits_ref):
    h = h_ref[...]
    w = w_ref[...]
    logits = jax.lax.dot_general(
        w, h, (((1,), (1,)), ((), ())), preferred_element_type=jnp.float32
    )

    m = jnp.max(logits, axis=0, keepdims=True)
    e = jnp.exp(logits - m)
    probs = e / jnp.sum(e, axis=0, keepdims=True)

    row = jax.lax.broadcasted_iota(jnp.int32, logits.shape, 0)
    idx1 = jnp.min(jnp.where(logits == m, row, N_EXP), axis=0, keepdims=True)
    mask1 = row == idx1
    l2 = jnp.where(mask1, -jnp.inf, logits)
    m2 = jnp.max(l2, axis=0, keepdims=True)
    idx2 = jnp.min(jnp.where(l2 == m2, row, N_EXP), axis=0, keepdims=True)
    mask = mask1 | (row == idx2)

    sl = pl.ds(pl.program_id(0) * TILE, TILE)
    mask_ref[:, sl] = mask
    probs_ref[:, sl] = probs
    logits_ref[:, sl] = logits


@functools.partial(jax.jit, static_argnames=())
def kernel(h, W):
    n_tok = h.shape[0]
    grid = (n_tok // TILE,)
    out_shapes = (
        jax.ShapeDtypeStruct((N_EXP, n_tok), jnp.bool_),
        jax.ShapeDtypeStruct((N_EXP, n_tok), jnp.float32),
        jax.ShapeDtypeStruct((N_EXP, n_tok), jnp.float32),
    )
    out_spec = pl.BlockSpec(memory_space=pltpu.MemorySpace.VMEM)
    mask_t, probs_t, logits_t = pl.pallas_call(
        _router_kernel,
        grid=grid,
        in_specs=[
            pl.BlockSpec((TILE, D_MODEL), lambda i: (i, 0)),
            pl.BlockSpec((N_EXP, D_MODEL), lambda i: (0, 0)),
        ],
        out_specs=(out_spec, out_spec, out_spec),
        out_shape=out_shapes,
        compiler_params=pltpu.CompilerParams(
            dimension_semantics=("parallel",),
        ),
    )(h, W)
    return mask_t.T, probs_t.T, logits_t.T
